# Initial kernel scaffold; baseline (speedup 1.0000x reference)
import jax
import jax.numpy as jnp
from jax.experimental import pallas as pl


def kernel(x, Wq, bq, Wk, bk, Wv, bv, W, b):
    raise NotImplementedError("write your pallas kernel here")


# speedup vs baseline: 124.8894x; 124.8894x over previous
import functools

import jax
import jax.numpy as jnp
from jax.experimental import pallas as pl


def _proj_kernel(x_ref, wq_ref, bq_ref, wk_ref, bk_ref, wv_ref, bv_ref,
                 w_ref, q_ref, kk_ref, sup_ref):
    x = x_ref[0]
    dn = (((1,), (1,)), ((), ()))
    q = jax.lax.dot_general(x, wq_ref[...], dn,
                            preferred_element_type=jnp.float32) + bq_ref[...]
    kk = jax.lax.dot_general(x, wk_ref[...], dn,
                             preferred_element_type=jnp.float32) + bk_ref[...]
    v = jax.lax.dot_general(x, wv_ref[...], dn,
                            preferred_element_type=jnp.float32) + bv_ref[...]
    sup = jnp.dot(v, w_ref[...], preferred_element_type=jnp.float32)
    q_ref[0] = q
    kk_ref[0] = kk
    sup_ref[0] = sup


def _attn_kernel(k_top, kk_ref, q_ref, sup_ref, b_ref, o_ref):
    kk = kk_ref[0]
    q = q_ref[0]
    L = jax.lax.dot_general(kk, q, (((1,), (1,)), ((), ())),
                            preferred_element_type=jnp.float32)
    m = jnp.max(L, axis=-1, keepdims=True)
    e = jnp.exp(L - m)
    z = jnp.sum(e, axis=-1, keepdims=True)

    xi = jax.lax.bitcast_convert_type(L, jnp.int32)
    ki = xi ^ ((xi >> 31) & jnp.int32(0x7FFFFFFF))
    lo0 = jnp.min(ki, axis=-1, keepdims=True)
    hi0 = jnp.max(ki, axis=-1, keepdims=True) + 1

    def body(_, carry):
        lo, hi = carry
        mid = (lo >> 1) + (hi >> 1) + (lo & hi & 1)
        cnt = jnp.sum((ki >= mid).astype(jnp.int32), axis=-1, keepdims=True)
        ge = cnt >= k_top
        return jnp.where(ge, mid, lo), jnp.where(ge, hi, mid)

    lo, _ = jax.lax.fori_loop(0, 32, body, (lo0, hi0))
    mask = ki >= lo

    p = e / z
    w2 = jnp.where(mask, jnp.exp(p), 0.0)
    a2 = w2 / jnp.sum(w2, axis=-1, keepdims=True)
    o_ref[0] = jnp.dot(a2, sup_ref[0],
                       preferred_element_type=jnp.float32) + b_ref[...]


def kernel(x, Wq, bq, Wk, bk, Wv, bv, W, b):
    B, s, c = x.shape
    d = Wq.shape[0]
    out = W.shape[1]
    k_top = int(s / 3 * 2)
    R = 256

    f32 = jnp.float32
    q, kk, sup = pl.pallas_call(
        _proj_kernel,
        grid=(B,),
        in_specs=[
            pl.BlockSpec((1, s, c), lambda i: (i, 0, 0)),
            pl.BlockSpec((d, c), lambda i: (0, 0)),
            pl.BlockSpec((1, d), lambda i: (0, 0)),
            pl.BlockSpec((d, c), lambda i: (0, 0)),
            pl.BlockSpec((1, d), lambda i: (0, 0)),
            pl.BlockSpec((out, c), lambda i: (0, 0)),
            pl.BlockSpec((1, out), lambda i: (0, 0)),
            pl.BlockSpec((c, out), lambda i: (0, 0)),
        ],
        out_specs=[
            pl.BlockSpec((1, s, d), lambda i: (i, 0, 0)),
            pl.BlockSpec((1, s, d), lambda i: (i, 0, 0)),
            pl.BlockSpec((1, s, out), lambda i: (i, 0, 0)),
        ],
        out_shape=[
            jax.ShapeDtypeStruct((B, s, d), f32),
            jax.ShapeDtypeStruct((B, s, d), f32),
            jax.ShapeDtypeStruct((B, s, out), f32),
        ],
    )(x, Wq, bq.reshape(1, d), Wk, bk.reshape(1, d), Wv, bv.reshape(1, out), W)

    y = pl.pallas_call(
        functools.partial(_attn_kernel, k_top),
        grid=(B, s // R),
        in_specs=[
            pl.BlockSpec((1, R, d), lambda i, j: (i, j, 0)),
            pl.BlockSpec((1, s, d), lambda i, j: (i, 0, 0)),
            pl.BlockSpec((1, s, out), lambda i, j: (i, 0, 0)),
            pl.BlockSpec((1, out), lambda i, j: (0, 0)),
        ],
        out_specs=pl.BlockSpec((1, R, out), lambda i, j: (i, j, 0)),
        out_shape=jax.ShapeDtypeStruct((B, s, out), f32),
    )(kk, q, sup, b.reshape(1, out))
    return y


# speedup vs baseline: 148.5828x; 1.1897x over previous
import functools

import jax
import jax.numpy as jnp
from jax.experimental import pallas as pl


def _proj_kernel(x_ref, wq_ref, bq_ref, wk_ref, bk_ref, wv_ref, bv_ref,
                 w_ref, q_ref, kk_ref, sup_ref):
    x = x_ref[0]
    dn = (((1,), (1,)), ((), ()))
    q = jax.lax.dot_general(x, wq_ref[...], dn,
                            preferred_element_type=jnp.float32) + bq_ref[...]
    kk = jax.lax.dot_general(x, wk_ref[...], dn,
                             preferred_element_type=jnp.float32) + bk_ref[...]
    v = jax.lax.dot_general(x, wv_ref[...], dn,
                            preferred_element_type=jnp.float32) + bv_ref[...]
    sup = jnp.dot(v, w_ref[...], preferred_element_type=jnp.float32)
    q_ref[0] = q
    kk_ref[0] = kk
    sup_ref[0] = sup


def _attn_kernel(k_top, kk_ref, q_ref, sup_ref, b_ref, o_ref):
    kk = kk_ref[0]
    q = q_ref[0]
    L = jax.lax.dot_general(kk, q, (((1,), (1,)), ((), ())),
                            preferred_element_type=jnp.float32)
    m = jnp.max(L, axis=-1, keepdims=True)
    e = jnp.exp(L - m)
    z = jnp.sum(e, axis=-1, keepdims=True)

    xi = jax.lax.bitcast_convert_type(L, jnp.int32)
    ki = xi ^ ((xi >> 31) & jnp.int32(0x7FFFFFFF))
    R = L.shape[0]
    s = L.shape[1]
    lo0 = jnp.min(ki, axis=-1, keepdims=True)
    hi0 = jnp.max(ki, axis=-1, keepdims=True) + 1
    cl0 = jnp.full((R, 1), float(s), jnp.float32)
    ch0 = jnp.zeros((R, 1), jnp.float32)
    kf = jnp.float32(k_top)

    def key_to_f(kk):
        bits = jnp.where(kk >= 0, kk, kk ^ jnp.int32(0x7FFFFFFF))
        return jax.lax.bitcast_convert_type(bits, jnp.float32)

    def cond(st):
        i, lo, hi, cl, ch = st
        return jnp.logical_and(i < 48, jnp.any(lo + 1 < hi))

    def body(st):
        i, lo, hi, cl, ch = st
        lo_v = key_to_f(lo)
        hi_v = key_to_f(hi - 1)
        frac = (cl - kf) / jnp.maximum(cl - ch, 1.0)
        mv = jax.lax.bitcast_convert_type(lo_v + frac * (hi_v - lo_v),
                                          jnp.int32)
        mid_i = mv ^ ((mv >> 31) & jnp.int32(0x7FFFFFFF))
        mid_b = (lo >> 1) + (hi >> 1) + (lo & hi & 1)
        mid = jnp.where((i & 7) == 7, mid_b, mid_i)
        ms = jnp.maximum((hi >> 8) - (lo >> 8), 1)
        mid = jnp.clip(mid, lo + ms, hi - ms)
        mid = jnp.clip(mid, lo + 1, hi - 1)
        cnt = jnp.sum(jnp.where(ki >= mid, 1.0, 0.0), axis=-1, keepdims=True)
        ge = cnt >= kf
        is_k = cnt == kf
        nlo = jnp.where(ge, mid, lo)
        ncl = jnp.where(ge, cnt, cl)
        nhi = jnp.where(is_k, nlo + 1, jnp.where(ge, hi, mid))
        nch = jnp.where(ge, ch, cnt)
        return i + 1, nlo, nhi, ncl, nch

    _, lo, _, _, _ = jax.lax.while_loop(
        cond, body, (jnp.int32(0), lo0, hi0, cl0, ch0))
    mask = ki >= lo

    p = e / z
    w2 = jnp.where(mask, jnp.exp(p), 0.0)
    a2 = w2 / jnp.sum(w2, axis=-1, keepdims=True)
    o_ref[0] = jnp.dot(a2, sup_ref[0],
                       preferred_element_type=jnp.float32) + b_ref[...]


def kernel(x, Wq, bq, Wk, bk, Wv, bv, W, b):
    B, s, c = x.shape
    d = Wq.shape[0]
    out = W.shape[1]
    k_top = int(s / 3 * 2)
    R = 256

    f32 = jnp.float32
    q, kk, sup = pl.pallas_call(
        _proj_kernel,
        grid=(B,),
        in_specs=[
            pl.BlockSpec((1, s, c), lambda i: (i, 0, 0)),
            pl.BlockSpec((d, c), lambda i: (0, 0)),
            pl.BlockSpec((1, d), lambda i: (0, 0)),
            pl.BlockSpec((d, c), lambda i: (0, 0)),
            pl.BlockSpec((1, d), lambda i: (0, 0)),
            pl.BlockSpec((out, c), lambda i: (0, 0)),
            pl.BlockSpec((1, out), lambda i: (0, 0)),
            pl.BlockSpec((c, out), lambda i: (0, 0)),
        ],
        out_specs=[
            pl.BlockSpec((1, s, d), lambda i: (i, 0, 0)),
            pl.BlockSpec((1, s, d), lambda i: (i, 0, 0)),
            pl.BlockSpec((1, s, out), lambda i: (i, 0, 0)),
        ],
        out_shape=[
            jax.ShapeDtypeStruct((B, s, d), f32),
            jax.ShapeDtypeStruct((B, s, d), f32),
            jax.ShapeDtypeStruct((B, s, out), f32),
        ],
    )(x, Wq, bq.reshape(1, d), Wk, bk.reshape(1, d), Wv, bv.reshape(1, out), W)

    y = pl.pallas_call(
        functools.partial(_attn_kernel, k_top),
        grid=(B, s // R),
        in_specs=[
            pl.BlockSpec((1, R, d), lambda i, j: (i, j, 0)),
            pl.BlockSpec((1, s, d), lambda i, j: (i, 0, 0)),
            pl.BlockSpec((1, s, out), lambda i, j: (i, 0, 0)),
            pl.BlockSpec((1, out), lambda i, j: (0, 0)),
        ],
        out_specs=pl.BlockSpec((1, R, out), lambda i, j: (i, j, 0)),
        out_shape=jax.ShapeDtypeStruct((B, s, out), f32),
    )(kk, q, sup, b.reshape(1, out))
    return y
